# Initial kernel scaffold; baseline (speedup 1.0000x reference)
import jax
import jax.numpy as jnp
from jax.experimental import pallas as pl


def kernel(x, edge_index, W1, b1, W3, b3, W4, b4, W2, b2):
    raise NotImplementedError("write your pallas kernel here")


# speedup vs baseline: 22.0999x; 22.0999x over previous
import functools

import jax
import jax.numpy as jnp
from jax import lax
from jax.experimental import pallas as pl
from jax.experimental.pallas import tpu as pltpu
from jax.experimental.pallas import tpu_sc as plsc

_N = 10000
_NPAD = 10240
_E = 320000
_NC = 2
_NS = 16
_NW = _NC * _NS
_EPW = _E // _NW
_CH = 128
_NCH = -(-_EPW // _CH)
_PADE = _NCH * _CH - _EPW
_DUMP = _NPAD - _N
_RPS = _NPAD // _NS
_DEGW = 16


def _sc_scatter_rows(D):
    mesh = plsc.VectorSubcoreMesh(core_axis_name="c", subcore_axis_name="s")

    def body(y_hbm, srcw, dstw, zeros_hbm, out_hbm, src_v, dst_v, rows_v, accum, sem):
        c = lax.axis_index("c")
        s = lax.axis_index("s")
        wid = s * _NC + c
        pltpu.sync_copy(srcw.at[wid], src_v)
        pltpu.sync_copy(dstw.at[wid], dst_v)
        pltpu.sync_copy(zeros_hbm.at[pl.ds(s * _RPS, _RPS)],
                        accum.at[pl.ds(s * _RPS, _RPS)])
        plsc.subcore_barrier()

        def step(j, carry):
            pltpu.async_copy(y_hbm.at[src_v.at[j]], rows_v, sem).wait()
            pltpu.sync_copy(rows_v, accum.at[dst_v.at[j]], add=True)
            return carry

        lax.fori_loop(0, _NCH, step, 0)
        plsc.subcore_barrier()
        pltpu.sync_copy(accum.at[pl.ds(s * _RPS, _RPS)],
                        out_hbm.at[c, pl.ds(s * _RPS, _RPS)])

    return pl.kernel(
        body,
        out_type=jax.ShapeDtypeStruct((_NC, _NPAD, D), jnp.float32),
        mesh=mesh,
        compiler_params=pltpu.CompilerParams(use_tc_tiling_on_sc=False),
        scratch_types=[
            pltpu.VMEM((_NCH, _CH), jnp.int32),
            pltpu.VMEM((_NCH, _CH), jnp.int32),
            pltpu.VMEM((_CH, D), jnp.float32),
            pltpu.VMEM_SHARED((_NPAD, D), jnp.float32),
            pltpu.SemaphoreType.DMA,
        ],
    )


def _sc_degree():
    mesh = plsc.VectorSubcoreMesh(core_axis_name="c", subcore_axis_name="s")

    def body(ones_hbm, dstw, zeros_hbm, out_hbm, dst_v, ones_v, accum, sem):
        c = lax.axis_index("c")
        s = lax.axis_index("s")
        wid = s * _NC + c
        pltpu.sync_copy(dstw.at[wid], dst_v)
        pltpu.sync_copy(ones_hbm, ones_v)
        pltpu.sync_copy(zeros_hbm.at[pl.ds(s * _RPS, _RPS)],
                        accum.at[pl.ds(s * _RPS, _RPS)])
        plsc.subcore_barrier()

        def step(j, carry):
            pltpu.sync_copy(ones_v, accum.at[dst_v.at[j]], add=True)
            return carry

        lax.fori_loop(0, _NCH, step, 0)
        plsc.subcore_barrier()
        pltpu.sync_copy(accum.at[pl.ds(s * _RPS, _RPS)],
                        out_hbm.at[c, pl.ds(s * _RPS, _RPS)])

    return pl.kernel(
        body,
        out_type=jax.ShapeDtypeStruct((_NC, _NPAD, _DEGW), jnp.float32),
        mesh=mesh,
        compiler_params=pltpu.CompilerParams(use_tc_tiling_on_sc=False),
        scratch_types=[
            pltpu.VMEM((_NCH, _CH), jnp.int32),
            pltpu.VMEM((_CH, _DEGW), jnp.float32),
            pltpu.VMEM_SHARED((_NPAD, _DEGW), jnp.float32),
            pltpu.SemaphoreType.DMA,
        ],
    )


_ROWS_BLK = 1024
_GRID = _NPAD // _ROWS_BLK


def _dis_block(degp_ref):
    deg = degp_ref[0, :, 0:1] + degp_ref[1, :, 0:1] + 1.0
    return lax.rsqrt(deg)


def _tc_first(x, W, degp):
    Din, Dout = W.shape

    def body(x_ref, w_ref, degp_ref, o_ref):
        dis = _dis_block(degp_ref)
        o_ref[...] = dis * jnp.dot(x_ref[...], w_ref[...],
                                   preferred_element_type=jnp.float32)

    return pl.pallas_call(
        body,
        grid=(_GRID,),
        in_specs=[
            pl.BlockSpec((_ROWS_BLK, Din), lambda r: (r, 0)),
            pl.BlockSpec((Din, Dout), lambda r: (0, 0)),
            pl.BlockSpec((2, _ROWS_BLK, _DEGW), lambda r: (0, r, 0)),
        ],
        out_specs=pl.BlockSpec((_ROWS_BLK, Dout), lambda r: (r, 0)),
        out_shape=jax.ShapeDtypeStruct((_NPAD, Dout), jnp.float32),
    )(x, W, degp)


def _tc_mid(p, yprev, b, W, degp):
    Din, Dout = W.shape

    def body(p_ref, y_ref, b_ref, w_ref, degp_ref, o_ref):
        dis = _dis_block(degp_ref)
        acc = p_ref[0] + p_ref[1] + y_ref[...]
        h = jnp.maximum(dis * acc + b_ref[...], 0.0)
        o_ref[...] = dis * jnp.dot(h, w_ref[...],
                                   preferred_element_type=jnp.float32)

    return pl.pallas_call(
        body,
        grid=(_GRID,),
        in_specs=[
            pl.BlockSpec((2, _ROWS_BLK, Din), lambda r: (0, r, 0)),
            pl.BlockSpec((_ROWS_BLK, Din), lambda r: (r, 0)),
            pl.BlockSpec((1, Din), lambda r: (0, 0)),
            pl.BlockSpec((Din, Dout), lambda r: (0, 0)),
            pl.BlockSpec((2, _ROWS_BLK, _DEGW), lambda r: (0, r, 0)),
        ],
        out_specs=pl.BlockSpec((_ROWS_BLK, Dout), lambda r: (r, 0)),
        out_shape=jax.ShapeDtypeStruct((_NPAD, Dout), jnp.float32),
    )(p, yprev, b.reshape(1, Din), W, degp)


def _tc_last(p, yprev, b, degp):
    Din = yprev.shape[1]

    def body(p_ref, y_ref, b_ref, degp_ref, o_ref):
        dis = _dis_block(degp_ref)
        acc = p_ref[0] + p_ref[1] + y_ref[...]
        o_ref[...] = jnp.maximum(dis * acc + b_ref[...], 0.0)

    return pl.pallas_call(
        body,
        grid=(_GRID,),
        in_specs=[
            pl.BlockSpec((2, _ROWS_BLK, Din), lambda r: (0, r, 0)),
            pl.BlockSpec((_ROWS_BLK, Din), lambda r: (r, 0)),
            pl.BlockSpec((1, Din), lambda r: (0, 0)),
            pl.BlockSpec((2, _ROWS_BLK, _DEGW), lambda r: (0, r, 0)),
        ],
        out_specs=pl.BlockSpec((_ROWS_BLK, Din), lambda r: (r, 0)),
        out_shape=jax.ShapeDtypeStruct((_NPAD, Din), jnp.float32),
    )(p, yprev, b.reshape(1, Din), degp)


def kernel(x, edge_index, W1, b1, W3, b3, W4, b4, W2, b2):
    x_pad = jnp.pad(x, ((0, _NPAD - _N), (0, 0)))
    src2 = edge_index[0].reshape(_NW, _EPW)
    dst2 = edge_index[1].reshape(_NW, _EPW)
    pad_ids = jnp.arange(_NW * _PADE, dtype=jnp.int32)
    pad_src = (pad_ids * 97 % _N).reshape(_NW, _PADE)
    pad_dst = (_N + pad_ids % _DUMP).reshape(_NW, _PADE)
    srcw = jnp.concatenate([src2, pad_src], axis=1).reshape(_NW, _NCH, _CH)
    dstw = jnp.concatenate([dst2, pad_dst], axis=1).reshape(_NW, _NCH, _CH)

    ones_deg = jnp.ones((_CH, _DEGW), jnp.float32)
    zeros = {d: jnp.zeros((_NPAD, d), jnp.float32) for d in (16, 32, 64, 128)}

    degp = _sc_degree()(ones_deg, dstw, zeros[_DEGW])

    y1 = _tc_first(x_pad, W1, degp)
    p1 = _sc_scatter_rows(128)(y1, srcw, dstw, zeros[128])
    y2 = _tc_mid(p1, y1, b1, W3, degp)
    p2 = _sc_scatter_rows(64)(y2, srcw, dstw, zeros[64])
    y3 = _tc_mid(p2, y2, b3, W4, degp)
    p3 = _sc_scatter_rows(32)(y3, srcw, dstw, zeros[32])
    y4 = _tc_mid(p3, y3, b4, W2, degp)
    p4 = _sc_scatter_rows(16)(y4, srcw, dstw, zeros[16])
    out = _tc_last(p4, y4, b2, degp)
    return out[:_N]
